# baseline (speedup 1.0000x reference)
import functools

import jax
import jax.numpy as jnp
import numpy as np
from jax import lax
from jax.experimental import pallas as pl
from jax.experimental.pallas import tpu as pltpu
from jax.experimental.pallas import tpu_sc as plsc

N_NODES = 10000
N_EDGES = 320000
D_IN = 128
D_HID = 16
D_OUT = 40

NC = 2
NS = 16
NW = NC * NS
LW = 128
R = 80
PADE = NW * R * LW
NPAD = 10112
ROWS_PT = NPAD // NS

_mesh = plsc.VectorSubcoreMesh(core_axis_name="c", subcore_axis_name="s")


def _zero_accum_slice(zbuf_v, accum_sh, base):
    for i in range(LW):
        zbuf_v[i, :] = jnp.zeros((16,), jnp.float32)
    full, rem = divmod(ROWS_PT, LW)
    for k in range(full):
        pltpu.sync_copy(zbuf_v.at[pl.ds(0, LW)],
                        accum_sh.at[pl.ds(base + k * LW, LW)])
    if rem:
        pltpu.sync_copy(zbuf_v.at[pl.ds(0, rem)],
                        accum_sh.at[pl.ds(base + full * LW, rem)])


NBUF = 8


def _deg_body(edge_hbm, out_hbm, dstidx_v, ones_v, obuf_v, *rest):
    ssems = rest[:NBUF]
    accum_sh = rest[NBUF]
    c = lax.axis_index("c")
    s = lax.axis_index("s")
    tile = c * NS + s
    base = pl.multiple_of(s * ROWS_PT, 8)
    rows = pl.ds(base, ROWS_PT)
    pltpu.sync_copy(edge_hbm.at[1, tile], dstidx_v)
    for i in range(LW):
        ones_v[i, :] = jnp.full((16,), 1.0, jnp.float32)
    _zero_accum_slice(obuf_v, accum_sh, base)
    plsc.subcore_barrier()

    def body(i, carry):
        base_j = i * NBUF
        descs = [
            pltpu.async_copy(
                ones_v, accum_sh.at[dstidx_v.at[base_j + k]], ssems[k],
                add=True)
            for k in range(NBUF)
        ]
        for d in descs:
            d.wait()
        return carry

    lax.fori_loop(0, R // NBUF, body, 0)
    plsc.subcore_barrier()
    pltpu.sync_copy(accum_sh.at[rows], obuf_v)
    pltpu.sync_copy(obuf_v, out_hbm.at[c, rows])


def _scat_body(g_hbm, edge_hbm, out_hbm,
               srcidx_v, dstidx_v, obuf_v, *rest):
    bufs = rest[:NBUF]
    gsems = rest[NBUF:2 * NBUF]
    ssems = rest[2 * NBUF:3 * NBUF]
    g_sh, accum_sh = rest[3 * NBUF], rest[3 * NBUF + 1]
    c = lax.axis_index("c")
    s = lax.axis_index("s")
    tile = c * NS + s
    base = pl.multiple_of(s * ROWS_PT, 8)
    rows = pl.ds(base, ROWS_PT)
    pltpu.sync_copy(edge_hbm.at[0, tile], srcidx_v)
    pltpu.sync_copy(edge_hbm.at[1, tile], dstidx_v)
    pltpu.sync_copy(g_hbm.at[rows], obuf_v)
    pltpu.sync_copy(obuf_v, g_sh.at[rows])
    _zero_accum_slice(bufs[0], accum_sh, base)
    plsc.subcore_barrier()

    def body(i, carry):
        base_j = i * NBUF
        gd = [
            pltpu.async_copy(
                g_sh.at[srcidx_v.at[base_j + k]], bufs[k], gsems[k])
            for k in range(NBUF)
        ]
        sd = []
        for k in range(NBUF):
            gd[k].wait()
            sd.append(pltpu.async_copy(
                bufs[k], accum_sh.at[dstidx_v.at[base_j + k]], ssems[k],
                add=True))
        for d in sd:
            d.wait()
        return carry

    lax.fori_loop(0, R // NBUF, body, 0)
    plsc.subcore_barrier()
    pltpu.sync_copy(accum_sh.at[rows], obuf_v)
    pltpu.sync_copy(obuf_v, out_hbm.at[c, rows])


_sc_params = pltpu.CompilerParams(use_tc_tiling_on_sc=False)

_deg_kernel = functools.partial(
    pl.kernel,
    out_type=jax.ShapeDtypeStruct((NC, NPAD, 16), jnp.float32),
    mesh=_mesh,
    compiler_params=_sc_params,
    scratch_types=[
        pltpu.VMEM((R, LW), jnp.int32),
        pltpu.VMEM((LW, 16), jnp.float32),
        pltpu.VMEM((ROWS_PT, 16), jnp.float32),
    ] + [pltpu.SemaphoreType.DMA] * NBUF + [
        pltpu.VMEM_SHARED((NPAD, 16), jnp.float32),
    ],
)(_deg_body)

_scat_kernel = functools.partial(
    pl.kernel,
    out_type=jax.ShapeDtypeStruct((NC, NPAD, 16), jnp.float32),
    mesh=_mesh,
    compiler_params=_sc_params,
    scratch_types=[
        pltpu.VMEM((R, LW), jnp.int32),
        pltpu.VMEM((R, LW), jnp.int32),
        pltpu.VMEM((ROWS_PT, 16), jnp.float32),
    ] + [pltpu.VMEM((LW, 16), jnp.float32)] * NBUF
    + [pltpu.SemaphoreType.DMA] * (2 * NBUF) + [
        pltpu.VMEM_SHARED((NPAD, 16), jnp.float32),
        pltpu.VMEM_SHARED((NPAD, 16), jnp.float32),
    ],
)(_scat_body)


WROWS = NPAD // 8


def _to_wide(a):
    return jnp.concatenate(
        [a[k * WROWS:(k + 1) * WROWS] for k in range(8)], axis=1)


def _to_rows(a):
    return jnp.concatenate(
        [a[:, k * D_HID:(k + 1) * D_HID] for k in range(8)], axis=0)


def _dinv_wide(cnt_ref):
    cnt = cnt_ref[0] + cnt_ref[1]
    return lax.rsqrt(cnt + 1.0)


def _tc1_body(x_ref, w_ref, cnt_ref, g_ref):
    h = jnp.dot(x_ref[...], w_ref[...], preferred_element_type=jnp.float32)
    hp = jnp.concatenate(
        [h, jnp.zeros((NPAD - N_NODES, D_HID), jnp.float32)], axis=0)
    g_ref[...] = _to_wide(hp) * _dinv_wide(cnt_ref)


def _tc2_body(s_ref, g_ref, cnt_ref, b_ref, o_ref):
    dinv = _dinv_wide(cnt_ref)
    agg = dinv * (s_ref[0] + s_ref[1] + g_ref[...])
    o_ref[...] = jnp.maximum(agg + b_ref[...], 0.0) * dinv


def _tc3_body(s_ref, g_ref, cnt_ref, w_ref, b_ref, o_ref):
    dinv = _dinv_wide(cnt_ref)
    agg = dinv * (s_ref[0] + s_ref[1] + g_ref[...])
    aggn = _to_rows(agg)[:N_NODES]
    o_ref[...] = (
        jnp.dot(aggn, w_ref[...], preferred_element_type=jnp.float32)
        + b_ref[...])


_PAD_FILL = (lambda f: (f % WROWS) * 8 + f // WROWS)(
    N_NODES + (np.arange(PADE - N_EDGES, dtype=np.int32) % (NPAD - N_NODES))
).astype(np.int32)


def kernel(train_data, train_edge_index, training, W1, b1, W2, b2):
    del training
    eip = (train_edge_index % WROWS) * 8 + train_edge_index // WROWS
    fill2 = jnp.broadcast_to(jnp.asarray(_PAD_FILL), (2, _PAD_FILL.shape[0]))
    edges = jnp.concatenate([eip, fill2], axis=1).reshape(2, NW, R, LW)

    cnt = _deg_kernel(edges).reshape(NC, WROWS, 128)

    g1 = pl.pallas_call(
        _tc1_body,
        out_shape=jax.ShapeDtypeStruct((WROWS, 128), jnp.float32),
    )(train_data, W1, cnt)

    s1 = _scat_kernel(g1.reshape(NPAD, D_HID), edges)

    g2 = pl.pallas_call(
        _tc2_body,
        out_shape=jax.ShapeDtypeStruct((WROWS, 128), jnp.float32),
    )(s1.reshape(NC, WROWS, 128), g1, cnt, jnp.tile(b1, 8).reshape(1, 128))

    s2 = _scat_kernel(g2.reshape(NPAD, D_HID), edges)

    out = pl.pallas_call(
        _tc3_body,
        out_shape=jax.ShapeDtypeStruct((N_NODES, D_OUT), jnp.float32),
    )(s2.reshape(NC, WROWS, 128), g2, cnt, W2, b2.reshape(1, D_OUT))
    return out


# speedup vs baseline: 1.1997x; 1.1997x over previous
import functools

import jax
import jax.numpy as jnp
import numpy as np
from jax import lax
from jax.experimental import pallas as pl
from jax.experimental.pallas import tpu as pltpu
from jax.experimental.pallas import tpu_sc as plsc

N_NODES = 10000
N_EDGES = 320000
D_IN = 128
D_HID = 16
D_OUT = 40

NC = 2
NS = 16
NW = NC * NS
LW = 128
R = 80
PADE = NW * R * LW
NPAD = 10112
ROWS_PT = NPAD // NS

_mesh = plsc.VectorSubcoreMesh(core_axis_name="c", subcore_axis_name="s")


def _zero_accum_slice(zbuf_v, accum_sh, base):
    for i in range(LW):
        zbuf_v[i, :] = jnp.zeros((16,), jnp.float32)
    full, rem = divmod(ROWS_PT, LW)
    for k in range(full):
        pltpu.sync_copy(zbuf_v.at[pl.ds(0, LW)],
                        accum_sh.at[pl.ds(base + k * LW, LW)])
    if rem:
        pltpu.sync_copy(zbuf_v.at[pl.ds(0, rem)],
                        accum_sh.at[pl.ds(base + full * LW, rem)])


NBUF = 8


def _deg_body(edge_hbm, out_hbm, dstidx_v, ones_v, obuf_v, *rest):
    ssems = rest[:NBUF]
    accum_sh = rest[NBUF]
    c = lax.axis_index("c")
    s = lax.axis_index("s")
    tile = c * NS + s
    base = pl.multiple_of(s * ROWS_PT, 8)
    rows = pl.ds(base, ROWS_PT)
    pltpu.sync_copy(edge_hbm.at[1, tile], dstidx_v)
    for i in range(LW):
        ones_v[i, :] = jnp.full((16,), 1.0, jnp.float32)
    _zero_accum_slice(obuf_v, accum_sh, base)
    plsc.subcore_barrier()

    def body(i, carry):
        base_j = i * NBUF
        descs = [
            pltpu.async_copy(
                ones_v, accum_sh.at[dstidx_v.at[base_j + k]], ssems[k],
                add=True)
            for k in range(NBUF)
        ]
        for d in descs:
            d.wait()
        return carry

    lax.fori_loop(0, R // NBUF, body, 0)
    plsc.subcore_barrier()
    pltpu.sync_copy(accum_sh.at[rows], obuf_v)
    pltpu.sync_copy(obuf_v, out_hbm.at[c, rows])


def _scat_body(g_hbm, edge_hbm, out_hbm,
               srcidx_v, dstidx_v, obuf_v, *rest):
    bufs = rest[:NBUF]
    gsems = rest[NBUF:2 * NBUF]
    ssems = rest[2 * NBUF:3 * NBUF]
    accum_sh = rest[3 * NBUF]
    c = lax.axis_index("c")
    s = lax.axis_index("s")
    tile = c * NS + s
    base = pl.multiple_of(s * ROWS_PT, 8)
    rows = pl.ds(base, ROWS_PT)
    pltpu.sync_copy(edge_hbm.at[0, tile], srcidx_v)
    pltpu.sync_copy(edge_hbm.at[1, tile], dstidx_v)
    _zero_accum_slice(bufs[0], accum_sh, base)
    plsc.subcore_barrier()

    def body(i, carry):
        base_j = i * NBUF
        gd = [
            pltpu.async_copy(
                g_hbm.at[srcidx_v.at[base_j + k]], bufs[k], gsems[k])
            for k in range(NBUF)
        ]
        sd = []
        for k in range(NBUF):
            gd[k].wait()
            sd.append(pltpu.async_copy(
                bufs[k], accum_sh.at[dstidx_v.at[base_j + k]], ssems[k],
                add=True))
        for d in sd:
            d.wait()
        return carry

    lax.fori_loop(0, R // NBUF, body, 0)
    plsc.subcore_barrier()
    pltpu.sync_copy(accum_sh.at[rows], obuf_v)
    pltpu.sync_copy(obuf_v, out_hbm.at[c, rows])


_sc_params = pltpu.CompilerParams(use_tc_tiling_on_sc=False)

_deg_kernel = functools.partial(
    pl.kernel,
    out_type=jax.ShapeDtypeStruct((NC, NPAD, 16), jnp.float32),
    mesh=_mesh,
    compiler_params=_sc_params,
    scratch_types=[
        pltpu.VMEM((R, LW), jnp.int32),
        pltpu.VMEM((LW, 16), jnp.float32),
        pltpu.VMEM((ROWS_PT, 16), jnp.float32),
    ] + [pltpu.SemaphoreType.DMA] * NBUF + [
        pltpu.VMEM_SHARED((NPAD, 16), jnp.float32),
    ],
)(_deg_body)

_scat_kernel = functools.partial(
    pl.kernel,
    out_type=jax.ShapeDtypeStruct((NC, NPAD, 16), jnp.float32),
    mesh=_mesh,
    compiler_params=_sc_params,
    scratch_types=[
        pltpu.VMEM((R, LW), jnp.int32),
        pltpu.VMEM((R, LW), jnp.int32),
        pltpu.VMEM((ROWS_PT, 16), jnp.float32),
    ] + [pltpu.VMEM((LW, 16), jnp.float32)] * NBUF
    + [pltpu.SemaphoreType.DMA] * (2 * NBUF) + [
        pltpu.VMEM_SHARED((NPAD, 16), jnp.float32),
    ],
)(_scat_body)


WROWS = NPAD // 8


def _to_wide(a):
    return jnp.concatenate(
        [a[k * WROWS:(k + 1) * WROWS] for k in range(8)], axis=1)


def _to_rows(a):
    return jnp.concatenate(
        [a[:, k * D_HID:(k + 1) * D_HID] for k in range(8)], axis=0)


def _dinv_wide(cnt_ref):
    cnt = cnt_ref[0] + cnt_ref[1]
    return lax.rsqrt(cnt + 1.0)


def _tc1_body(x_ref, w_ref, cnt_ref, g_ref):
    h = jnp.dot(x_ref[...], w_ref[...], preferred_element_type=jnp.float32)
    hp = jnp.concatenate(
        [h, jnp.zeros((NPAD - N_NODES, D_HID), jnp.float32)], axis=0)
    g_ref[...] = _to_wide(hp) * _dinv_wide(cnt_ref)


def _tc2_body(s_ref, g_ref, cnt_ref, b_ref, o_ref):
    dinv = _dinv_wide(cnt_ref)
    agg = dinv * (s_ref[0] + s_ref[1] + g_ref[...])
    o_ref[...] = jnp.maximum(agg + b_ref[...], 0.0) * dinv


def _tc3_body(s_ref, g_ref, cnt_ref, w_ref, b_ref, o_ref):
    dinv = _dinv_wide(cnt_ref)
    agg = dinv * (s_ref[0] + s_ref[1] + g_ref[...])
    aggn = _to_rows(agg)[:N_NODES]
    o_ref[...] = lax.dot_general(
        w_ref[...], aggn, (((0,), (1,)), ((), ())),
        preferred_element_type=jnp.float32) + b_ref[...]


_PAD_FILL = (lambda f: (f % WROWS) * 8 + f // WROWS)(
    N_NODES + (np.arange(PADE - N_EDGES, dtype=np.int32) % (NPAD - N_NODES))
).astype(np.int32)
_EROWS = N_EDGES // LW


def _edges_body(ei_ref, fill_ref, o_ref):
    n = ei_ref[...]
    q = jnp.floor(
        (n.astype(jnp.float32) + 0.5) * jnp.float32(1.0 / WROWS)
    ).astype(jnp.int32)
    o_ref[:, :_EROWS] = (n - q * WROWS) * 8 + q
    o_ref[:, _EROWS:] = fill_ref[...]


def kernel(train_data, train_edge_index, training, W1, b1, W2, b2):
    del training
    fill2 = jnp.broadcast_to(
        jnp.asarray(_PAD_FILL).reshape(-1, LW),
        (2, (PADE - N_EDGES) // LW, LW))
    edges = pl.pallas_call(
        _edges_body,
        out_shape=jax.ShapeDtypeStruct((2, PADE // LW, LW), jnp.int32),
    )(train_edge_index.reshape(2, _EROWS, LW), fill2).reshape(2, NW, R, LW)

    cnt = _deg_kernel(edges).reshape(NC, WROWS, 128)

    g1 = pl.pallas_call(
        _tc1_body,
        out_shape=jax.ShapeDtypeStruct((WROWS, 128), jnp.float32),
    )(train_data, W1, cnt)

    s1 = _scat_kernel(g1.reshape(NPAD, D_HID), edges)

    g2 = pl.pallas_call(
        _tc2_body,
        out_shape=jax.ShapeDtypeStruct((WROWS, 128), jnp.float32),
    )(s1.reshape(NC, WROWS, 128), g1, cnt, jnp.tile(b1, 8).reshape(1, 128))

    s2 = _scat_kernel(g2.reshape(NPAD, D_HID), edges)

    out_t = pl.pallas_call(
        _tc3_body,
        out_shape=jax.ShapeDtypeStruct((D_OUT, N_NODES), jnp.float32),
    )(s2.reshape(NC, WROWS, 128), g2, cnt, W2, b2.reshape(D_OUT, 1))
    return out_t.T
